# baseline (speedup 1.0000x reference)
import jax
import jax.numpy as jnp
from jax import lax
from jax.experimental import pallas as pl
from jax.experimental.pallas import tpu as pltpu
from jax.experimental.pallas import tpu_sc as plsc

_N = 10000
_E = 320000
_D = 128
_HID = 145
_HP = 256
_V = 3
_NC = 2
_NS = 16
_NW = _NC * _NS
_EPW = 10240
_EP = _NW * _EPW
_CR = 64
_CS = 64
_SR = 2
_S = 5
_GA_S = 3
_NCHR = _EPW // _CR
_NCHS = _EPW // _CS
_HG = 160 // 16
_BR = 1000
_GB = _N // _BR
_ZR = 40
_NZ = _N // _ZR
_NZT = -(-_NZ // _NS)
_EPS = 1e-15


def _tc_norm(x):
    return jnp.clip(jnp.sqrt(jnp.sum(x * x, axis=-1, keepdims=True)), _EPS, None)


def _tc_expmap0(u):
    n = _tc_norm(u)
    return jnp.tanh(n) * u / n


def _tc_logmap0(x):
    n = jnp.clip(_tc_norm(x), _EPS, 1.0 - 1e-5)
    return 0.5 * jnp.log((1.0 + n) / (1.0 - n)) * x / n


def _tc_mobius_add(x, y):
    x2 = jnp.sum(x * x, axis=-1, keepdims=True)
    y2 = jnp.sum(y * y, axis=-1, keepdims=True)
    xy = jnp.sum(x * y, axis=-1, keepdims=True)
    num = (1.0 + 2.0 * xy + y2) * x + (1.0 - x2) * y
    den = jnp.clip(1.0 + 2.0 * xy + x2 * y2, _EPS, None)
    return num / den


def _leaky(x):
    return jnp.where(x >= 0, x, 0.1 * x)


def _tc1_body(x_ref, ca_ref, tab_ref, cid_ref,
              w1a_ref, w1b_ref, w1c_ref, u_ref, t_ref, b1_ref, lin1_ref,
              ps_ref, pd_ref, y1_ref, xh_ref):
    xb = x_ref[...]
    ca = ca_ref[...]
    idx = cid_ref[0]
    onehot = (lax.broadcasted_iota(jnp.int32, (16, 1), 0) == idx
              ).astype(jnp.float32)
    ctx = jnp.sum(onehot * tab_ref[...], axis=0, keepdims=True)
    xh = _tc_expmap0(xb)
    xh_ref[...] = xh
    xtan = _tc_logmap0(xh)
    for v in range(_V):
        ps_ref[v] = (jnp.dot(xb, w1a_ref[v], preferred_element_type=jnp.float32)
                     + ca * u_ref[v])
        addv = (jnp.dot(ctx, w1c_ref[v], preferred_element_type=jnp.float32)
                + b1_ref[v])
        pd_ref[v] = (jnp.dot(xb, w1b_ref[v], preferred_element_type=jnp.float32)
                     + ca * t_ref[v] + addv)
        z = jnp.dot(xtan, lin1_ref[v], preferred_element_type=jnp.float32)
        y1_ref[v] = _tc_logmap0(_tc_expmap0(z))


def _tc2_body(pa0, pb0, pa1, pb1, pa2, pb2, y1_ref, xh_ref, lin2_ref,
              h1_ref, y2_ref):
    xh = xh_ref[...]
    pa = (pa0, pa1, pa2)
    pb = (pb0, pb1, pb2)
    for v in range(_V):
        agg = pa[v][...] + pb[v][...] + y1_ref[v]
        h = _tc_expmap0(_leaky(agg))
        h1 = _tc_mobius_add(h, xh)
        h1_ref[v] = h1
        z = jnp.dot(_tc_logmap0(h1), lin2_ref[v],
                    preferred_element_type=jnp.float32)
        y2_ref[v] = _tc_logmap0(_tc_expmap0(z))


def _tc3_body(pa0, pb0, pa1, pb1, pa2, pb2, y2_ref, h1_ref,
              out_ref, h2_ref):
    pa = (pa0, pa1, pa2)
    pb = (pb0, pb1, pb2)
    for v in range(_V):
        agg = pa[v][...] + pb[v][...] + y2_ref[v]
        h = _tc_expmap0(_leaky(agg))
        h2 = _tc_mobius_add(h, h1_ref[v])
        h2_ref[v] = h2
        out_ref[:, v * _D:(v + 1) * _D] = _tc_logmap0(h2)


def _rw_body(*refs):
    (pc0, pc1, pc2, ek0, ek1, ek2, wt0, wt1, wt2, b2p0, b2p1, b2p2,
     wo0, wo1, wo2) = refs[:15]
    r = list(refs[15:])
    cbuf = r[0:_SR]; pbuf = r[_SR:2 * _SR]; wbuf = r[2 * _SR:3 * _SR]
    w2t = r[3 * _SR]; b2b = r[3 * _SR + 1]
    sems = r[3 * _SR + 2:]
    gp = sems[0:_SR]; pk = sems[_SR:2 * _SR]; ws = sems[2 * _SR:3 * _SR]

    cid = lax.axis_index("c")
    sid = lax.axis_index("s")
    wid = sid * _NC + cid
    chunk0 = wid * _NCHR
    pcv_ = (pc0, pc1, pc2)
    ekv_ = (ek0, ek1, ek2)
    wtv_ = (wt0, wt1, wt2)
    b2v_ = (b2p0, b2p1, b2p2)
    wov_ = (wo0, wo1, wo2)
    egs = [lax.iota(jnp.int32, 16) + 16 * g for g in range(_CR // 16)]
    egd = [e + _CR for e in egs]
    d_pk = ekv_[0].at[pl.ds(0, 3 * _CR)]
    d_row = pcv_[0].at[pl.ds(0, 2 * _CR)]
    d_w = wov_[0].at[pl.ds(0, _CR)]

    def _wait(dummy, dst, sem):
        pltpu.make_async_copy(dummy, dst, sem).wait()

    for v in range(_V):
        pltpu.sync_copy(wtv_[v], w2t)
        pltpu.sync_copy(b2v_[v], b2b)
        b2l = b2b[...]

        def _issue_pack(c, j, v=v):
            pltpu.async_copy(
                ekv_[v].at[pl.ds((chunk0 + c) * 3 * _CR, 3 * _CR)],
                pbuf[j], pk[j])

        def _issue_gather(j, v=v):
            pltpu.async_copy(pcv_[v].at[pbuf[j].at[pl.ds(0, 2 * _CR)]],
                             cbuf[j], gp[j])

        for j in range(_SR):
            _issue_pack(j, j)
        _wait(d_pk, pbuf[0], pk[0])
        _issue_gather(0)

        def _group(g, _, v=v):
            for j in range(_SR):
                c = g * _SR + j
                j2 = 1 - j
                _wait(d_row, cbuf[j], gp[j])
                @pl.when(c + 1 < _NCHR)
                def _():
                    _wait(d_pk, pbuf[j2], pk[j2])
                    _issue_gather(j2)

                def _cg(cg, accs):
                    bcv = jnp.full((16,), cg * 16, jnp.int32)
                    o = list(accs)
                    for k in range(16):
                        colv = bcv + k
                        w2c = w2t[cg * 16 + k]
                        for gg in range(_CR // 16):
                            sg = plsc.load_gather(cbuf[j], [egs[gg], colv])
                            dg = plsc.load_gather(cbuf[j], [egd[gg], colv])
                            o[gg] = o[gg] + jnp.maximum(sg + dg, 0.0) * w2c
                    return tuple(o)
                accs = lax.fori_loop(
                    0, _HG, _cg,
                    tuple(jnp.zeros((16,), jnp.float32)
                          for _ in range(_CR // 16)))
                @pl.when(c >= _SR)
                def _():
                    _wait(d_w, wbuf[j], ws[j])
                for gg in range(_CR // 16):
                    t = accs[gg] + b2l
                    dyn = 1.0 / (1.0 + jnp.exp(-t))
                    scg = plsc.bitcast(
                        pbuf[j][pl.ds(2 * _CR + gg * 16, 16)], jnp.float32)
                    wbuf[j][pl.ds(gg * 16, 16)] = scg * dyn
                base = wid * _EPW + c * _CR
                pltpu.async_copy(wbuf[j], wov_[v].at[pl.ds(base, _CR)], ws[j])
                @pl.when(c + _SR < _NCHR)
                def _():
                    _issue_pack(c + _SR, j)
            return 0
        lax.fori_loop(0, _NCHR // _SR, _group, 0)
        for j in range(_SR):
            _wait(d_w, wbuf[j], ws[j])


def _scat_body(*refs):
    (ya, yb, yc, ek0, ek1, ek2, w0, w1, w2, p0, p1, p2) = refs[:12]
    r = list(refs[12:])
    ybuf = r[0:_S]; pbuf = r[_S:2 * _S]; wbuf = r[2 * _S:3 * _S]
    accsh = r[3 * _S]
    sems = r[3 * _S + 1:]
    gy = sems[0:_S]; pk = sems[_S:2 * _S]; iww = sems[2 * _S:3 * _S]
    scs = sems[3 * _S:4 * _S]; zs = sems[4 * _S]

    cid = lax.axis_index("c")
    sid = lax.axis_index("s")
    wid = sid * _NC + cid
    base0 = wid * _EPW
    chunk0 = wid * _NCHS
    yv_ = (ya, yb, yc)
    ekv_ = (ek0, ek1, ek2)
    wv_ = (w0, w1, w2)
    pv_ = (p0, p1, p2)
    zv = jnp.zeros((16,), jnp.float32)
    egs = [lax.iota(jnp.int32, 16) + 16 * g for g in range(_CS // 16)]
    d_pk = ekv_[0].at[0]
    d_w = wv_[0].at[pl.ds(0, _CS)]
    d_row = yv_[0].at[pl.ds(0, _CS)]
    d_z = yv_[0].at[pl.ds(0, _ZR)]

    def _wait(dummy, dst, sem):
        pltpu.make_async_copy(dummy, dst, sem).wait()

    for v in range(_V):
        def _zrow(rr, _):
            for k in range(_D // 16):
                ybuf[0][rr, pl.ds(k * 16, 16)] = zv
            return 0
        lax.fori_loop(0, _ZR, _zrow, 0)
        for tt in range(_NZT):
            ch = sid + _NS * tt
            @pl.when(ch < _NZ)
            def _():
                pltpu.async_copy(ybuf[0].at[pl.ds(0, _ZR)],
                                 accsh.at[pl.ds(ch * _ZR, _ZR)], zs)
        for tt in range(_NZT):
            ch = sid + _NS * tt
            @pl.when(ch < _NZ)
            def _():
                _wait(d_z, ybuf[0].at[pl.ds(0, _ZR)], zs)
        plsc.subcore_barrier()

        def _issue_idx(c, j, v=v):
            pltpu.async_copy(ekv_[v].at[chunk0 + c], pbuf[j], pk[j])
            pltpu.async_copy(wv_[v].at[pl.ds(base0 + c * _CS, _CS)],
                             wbuf[j], iww[j])

        def _issue_gather(j, v=v):
            pltpu.async_copy(yv_[v].at[pbuf[j].at[0]], ybuf[j], gy[j])

        for j in range(_S):
            _issue_idx(j, j)
        for j in range(_GA_S):
            _wait(d_pk, pbuf[j], pk[j])
            _issue_gather(j)

        def _group(g, _, v=v):
            for j in range(_S):
                c = g * _S + j
                _wait(d_row, ybuf[j], gy[j])
                _wait(d_w, wbuf[j], iww[j])

                wg = [wbuf[j][pl.ds(gg * 16, 16)]
                      for gg in range(_CS // 16)]

                def _scol(cg, _):
                    bcv = jnp.full((16,), cg * 16, jnp.int32)
                    for k in range(16):
                        colv = bcv + k
                        for gg in range(_CS // 16):
                            yvv = plsc.load_gather(ybuf[j], [egs[gg], colv])
                            plsc.store_scatter(ybuf[j], [egs[gg], colv],
                                               yvv * wg[gg])
                    return 0
                lax.fori_loop(0, _D // 16, _scol, 0)
                pltpu.async_copy(ybuf[j], accsh.at[pbuf[j].at[1]], scs[j],
                                 add=True)
                jg = (j + _GA_S) % _S
                @pl.when(c >= _S - _GA_S)
                def _():
                    _wait(d_row, ybuf[jg], scs[jg])
                @pl.when(c + _GA_S < _NCHS)
                def _():
                    _wait(d_pk, pbuf[jg], pk[jg])
                    _issue_gather(jg)
                @pl.when(c + _S < _NCHS)
                def _():
                    _issue_idx(c + _S, j)
            return 0
        lax.fori_loop(0, _NCHS // _S, _group, 0)
        for dd in range(_S - _GA_S):
            j = (_NCHS - 1 - dd) % _S
            _wait(d_row, ybuf[j], scs[j])
        plsc.subcore_barrier()
        for tt in range(_NZT):
            ch = sid + _NS * tt
            @pl.when(ch < _NZ)
            def _():
                pltpu.sync_copy(accsh.at[pl.ds(ch * _ZR, _ZR)],
                                pv_[v].at[pl.ds(cid * _N + ch * _ZR, _ZR)])
        plsc.subcore_barrier()


def _full(i):
    return (0,) * i


def kernel(x_proj, edge_indices, edge_scores, cancer_type_id, causal_scores,
           cancer_table, rw_W1_0, rw_b1_0, rw_W2_0, rw_b2_0, lin1_0, lin2_0,
           rw_W1_1, rw_b1_1, rw_W2_1, rw_b2_1, lin1_1, lin2_1,
           rw_W1_2, rw_b1_2, rw_W2_2, rw_b2_2, lin1_2, lin2_2):
    f32 = jnp.float32
    W1 = jnp.stack([rw_W1_0, rw_W1_1, rw_W1_2])
    W1 = jnp.pad(W1, ((0, 0), (0, 0), (0, _HP - _HID)))
    b1 = jnp.pad(jnp.stack([rw_b1_0, rw_b1_1, rw_b1_2]),
                 ((0, 0), (0, _HP - _HID)))[:, None, :]
    W2 = jnp.pad(jnp.stack([rw_W2_0, rw_W2_1, rw_W2_2])[..., 0],
                 ((0, 0), (0, _HP - _HID)))
    b2 = jnp.broadcast_to(jnp.stack([rw_b2_0, rw_b2_1, rw_b2_2]), (_V, 16))
    w1a = W1[:, :_D]
    w1b = W1[:, _D:2 * _D]
    w1c = W1[:, 2 * _D:2 * _D + 32]
    uvec = (W1[:, 2 * _D + 32] + W1[:, 2 * _D + 34])[:, None, :]
    tvec = (W1[:, 2 * _D + 33] - W1[:, 2 * _D + 34])[:, None, :]
    lin1 = jnp.stack([lin1_0, lin1_1, lin1_2])
    lin2 = jnp.stack([lin2_0, lin2_1, lin2_2])
    cid = cancer_type_id.astype(jnp.int32)

    pad = _EP - _E
    se = jnp.pad(edge_indices[:, 0, :], ((0, 0), (0, pad)))
    de = jnp.pad(edge_indices[:, 1, :], ((0, 0), (0, pad)))
    sc = jnp.pad(edge_scores, ((0, 0), (0, pad)))

    full = lambda shape: pl.BlockSpec(shape, lambda i: _full(len(shape)))
    tc1 = pl.pallas_call(
        _tc1_body,
        grid=(_GB,),
        in_specs=[
            pl.BlockSpec((_BR, _D), lambda i: (i, 0)),
            pl.BlockSpec((_BR, 1), lambda i: (i, 0)),
            full((16, 32)),
            pl.BlockSpec(memory_space=pltpu.SMEM),
            full((_V, _D, _HP)),
            full((_V, _D, _HP)),
            full((_V, 32, _HP)),
            full((_V, 1, _HP)),
            full((_V, 1, _HP)),
            full((_V, 1, _HP)),
            full((_V, _D, _D)),
        ],
        out_specs=[
            pl.BlockSpec((_V, _BR, _HP), lambda i: (0, i, 0)),
            pl.BlockSpec((_V, _BR, _HP), lambda i: (0, i, 0)),
            pl.BlockSpec((_V, _BR, _D), lambda i: (0, i, 0)),
            pl.BlockSpec((_BR, _D), lambda i: (i, 0)),
        ],
        out_shape=[
            jax.ShapeDtypeStruct((_V, _N, _HP), f32),
            jax.ShapeDtypeStruct((_V, _N, _HP), f32),
            jax.ShapeDtypeStruct((_V, _N, _D), f32),
            jax.ShapeDtypeStruct((_N, _D), f32),
        ],
    )
    ps, pd, y1, xh = tc1(x_proj, causal_scores, cancer_table, cid,
                         w1a, w1b, w1c, uvec, tvec, b1, lin1)

    pre_cat = jnp.concatenate([ps, pd], axis=1)
    i32 = jnp.int32
    se_r = se.reshape(_V, _NW * _NCHR, _CR)
    de_r = de.reshape(_V, _NW * _NCHR, _CR)
    sc_r = lax.bitcast_convert_type(sc, i32).reshape(_V, _NW * _NCHR, _CR)
    ekR = jnp.concatenate([se_r, de_r + _N, sc_r], axis=2).reshape(_V, -1)
    w2t = jnp.broadcast_to(W2[:, :, None], (_V, _HP, 16))
    ekS = jnp.stack([se.reshape(_V, _NW * _NCHS, _CS),
                     de.reshape(_V, _NW * _NCHS, _CS)], axis=2)

    mesh = plsc.VectorSubcoreMesh(core_axis_name="c", subcore_axis_name="s")
    sc_params = pltpu.CompilerParams(needs_layout_passes=False)
    rw = pl.kernel(
        _rw_body,
        compiler_params=sc_params,
        out_type=[jax.ShapeDtypeStruct((_EP,), f32)] * 3,
        mesh=mesh,
        scratch_types=(
            [pltpu.VMEM((2 * _CR, _HP), f32)] * _SR
            + [pltpu.VMEM((3 * _CR,), i32)] * _SR
            + [pltpu.VMEM((_CR,), f32)] * _SR
            + [pltpu.VMEM((_HP, 16), f32),
               pltpu.VMEM((16,), f32)]
            + [pltpu.SemaphoreType.DMA] * (3 * _SR)
        ),
    )
    w0, w1_, w2_ = rw(pre_cat[0], pre_cat[1], pre_cat[2],
                      ekR[0], ekR[1], ekR[2],
                      w2t[0], w2t[1], w2t[2], b2[0], b2[1], b2[2])

    scat_scratch = (
        [pltpu.VMEM((_CS, _D), f32)] * _S
        + [pltpu.VMEM((2, _CS), i32)] * _S
        + [pltpu.VMEM((_CS,), f32)] * _S
        + [pltpu.VMEM_SHARED((_N, _D), f32)]
        + [pltpu.SemaphoreType.DMA] * (4 * _S + 1)
    )
    scat1 = pl.kernel(
        _scat_body,
        compiler_params=sc_params,
        out_type=[jax.ShapeDtypeStruct((2 * _N, _D), f32)] * 3,
        mesh=mesh,
        scratch_types=scat_scratch,
    )
    q0, q1, q2 = scat1(y1[0], y1[1], y1[2],
                       ekS[0], ekS[1], ekS[2],
                       w0, w1_, w2_)

    half_a = pl.BlockSpec((_BR, _D), lambda i: (i, 0))
    half_b = pl.BlockSpec((_BR, _D), lambda i: (i + _GB, 0))
    tc2 = pl.pallas_call(
        _tc2_body,
        grid=(_GB,),
        in_specs=[half_a, half_b, half_a, half_b, half_a, half_b,
                  pl.BlockSpec((_V, _BR, _D), lambda i: (0, i, 0)),
                  pl.BlockSpec((_BR, _D), lambda i: (i, 0)),
                  full((_V, _D, _D))],
        out_specs=[pl.BlockSpec((_V, _BR, _D), lambda i: (0, i, 0)),
                   pl.BlockSpec((_V, _BR, _D), lambda i: (0, i, 0))],
        out_shape=[jax.ShapeDtypeStruct((_V, _N, _D), f32),
                   jax.ShapeDtypeStruct((_V, _N, _D), f32)],
    )
    h1, y2 = tc2(q0, q0, q1, q1, q2, q2, y1, xh, lin2)

    scat2 = pl.kernel(
        _scat_body,
        compiler_params=sc_params,
        out_type=[jax.ShapeDtypeStruct((2 * _N, _D), f32)] * 3,
        mesh=mesh,
        scratch_types=scat_scratch,
    )
    r0, r1, r2 = scat2(y2[0], y2[1], y2[2],
                       ekS[0], ekS[1], ekS[2],
                       w0, w1_, w2_)

    tc3 = pl.pallas_call(
        _tc3_body,
        grid=(_GB,),
        in_specs=[half_a, half_b, half_a, half_b, half_a, half_b,
                  pl.BlockSpec((_V, _BR, _D), lambda i: (0, i, 0)),
                  pl.BlockSpec((_V, _BR, _D), lambda i: (0, i, 0))],
        out_specs=[pl.BlockSpec((_BR, _V * _D), lambda i: (i, 0)),
                   pl.BlockSpec((_V, _BR, _D), lambda i: (0, i, 0))],
        out_shape=[jax.ShapeDtypeStruct((_N, _V * _D), f32),
                   jax.ShapeDtypeStruct((_V, _N, _D), f32)],
    )
    out, h2 = tc3(r0, r0, r1, r1, r2, r2, y2, h1)
    return (out, h2[0], h2[1], h2[2])


# speedup vs baseline: 3.0268x; 3.0268x over previous
import jax
import jax.numpy as jnp
from jax import lax
from jax.experimental import pallas as pl
from jax.experimental.pallas import tpu as pltpu
from jax.experimental.pallas import tpu_sc as plsc

_N = 10000
_E = 320000
_D = 128
_HID = 145
_HP = 256
_V = 3
_NC = 2
_NS = 16
_NW = _NC * _NS
_EPW = 10240
_EP = _NW * _EPW
_CR = 64
_CS = 64
_SR = 2
_S = 5
_GA_S = 3
_NCHR = _EPW // _CR
_NCHS = _EPW // _CS
_HG = 160 // 16
_BR = 1000
_GB = _N // _BR
_ZR = 40
_NZ = _N // _ZR
_NZT = -(-_NZ // _NS)
_EPS = 1e-15


def _tc_norm(x):
    return jnp.clip(jnp.sqrt(jnp.sum(x * x, axis=-1, keepdims=True)), _EPS, None)


def _tc_expmap0(u):
    n = _tc_norm(u)
    return jnp.tanh(n) * u / n


def _tc_logmap0(x):
    n = jnp.clip(_tc_norm(x), _EPS, 1.0 - 1e-5)
    return 0.5 * jnp.log((1.0 + n) / (1.0 - n)) * x / n


def _tc_mobius_add(x, y):
    x2 = jnp.sum(x * x, axis=-1, keepdims=True)
    y2 = jnp.sum(y * y, axis=-1, keepdims=True)
    xy = jnp.sum(x * y, axis=-1, keepdims=True)
    num = (1.0 + 2.0 * xy + y2) * x + (1.0 - x2) * y
    den = jnp.clip(1.0 + 2.0 * xy + x2 * y2, _EPS, None)
    return num / den


def _leaky(x):
    return jnp.where(x >= 0, x, 0.1 * x)


def _tc1_body(x_ref, ca_ref, tab_ref, cid_ref,
              w1a_ref, w1b_ref, w1c_ref, u_ref, t_ref, b1_ref, lin1_ref,
              ps_ref, pd_ref, y1_ref, xh_ref):
    xb = x_ref[...]
    ca = ca_ref[...]
    idx = cid_ref[0]
    onehot = (lax.broadcasted_iota(jnp.int32, (16, 1), 0) == idx
              ).astype(jnp.float32)
    ctx = jnp.sum(onehot * tab_ref[...], axis=0, keepdims=True)
    xh = _tc_expmap0(xb)
    xh_ref[...] = xh
    xtan = _tc_logmap0(xh)
    for v in range(_V):
        ps_ref[v] = (jnp.dot(xb, w1a_ref[v], preferred_element_type=jnp.float32)
                     + ca * u_ref[v])
        addv = (jnp.dot(ctx, w1c_ref[v], preferred_element_type=jnp.float32)
                + b1_ref[v])
        pd_ref[v] = (jnp.dot(xb, w1b_ref[v], preferred_element_type=jnp.float32)
                     + ca * t_ref[v] + addv)
        z = jnp.dot(xtan, lin1_ref[v], preferred_element_type=jnp.float32)
        y1_ref[v] = _tc_logmap0(_tc_expmap0(z))


def _tc2_body(pa0, pb0, pa1, pb1, pa2, pb2, y1_ref, xh_ref, lin2_ref,
              h1_ref, y2_ref):
    xh = xh_ref[...]
    pa = (pa0, pa1, pa2)
    pb = (pb0, pb1, pb2)
    for v in range(_V):
        agg = pa[v][...] + pb[v][...] + y1_ref[v]
        h = _tc_expmap0(_leaky(agg))
        h1 = _tc_mobius_add(h, xh)
        h1_ref[v] = h1
        z = jnp.dot(_tc_logmap0(h1), lin2_ref[v],
                    preferred_element_type=jnp.float32)
        y2_ref[v] = _tc_logmap0(_tc_expmap0(z))


def _tc3_body(pa0, pb0, pa1, pb1, pa2, pb2, y2_ref, h1_ref,
              out_ref, h2_ref):
    pa = (pa0, pa1, pa2)
    pb = (pb0, pb1, pb2)
    for v in range(_V):
        agg = pa[v][...] + pb[v][...] + y2_ref[v]
        h = _tc_expmap0(_leaky(agg))
        h2 = _tc_mobius_add(h, h1_ref[v])
        h2_ref[v] = h2
        out_ref[:, v * _D:(v + 1) * _D] = _tc_logmap0(h2)


_COLC = None


def _rw_body(*refs):
    global _COLC
    _COLC = [jnp.full((16,), kk, jnp.int32) for kk in range(17)]
    (pc0, pc1, pc2, ek0, ek1, ek2, wt0, wt1, wt2, b2p0, b2p1, b2p2,
     wo0, wo1, wo2) = refs[:15]
    r = list(refs[15:])
    cbuf = r[0:_SR]; pbuf = r[_SR:2 * _SR]; wbuf = r[2 * _SR:3 * _SR]
    w2b = r[3 * _SR]; b2b = r[3 * _SR + 1]; tbuf = r[3 * _SR + 2]
    sems = r[3 * _SR + 3:]
    gp = sems[0:_SR]; pk = sems[_SR:2 * _SR]; ws = sems[2 * _SR:3 * _SR]

    cid = lax.axis_index("c")
    sid = lax.axis_index("s")
    wid = sid * _NC + cid
    chunk0 = wid * _NCHR
    pcv_ = (pc0, pc1, pc2)
    ekv_ = (ek0, ek1, ek2)
    wtv_ = (wt0, wt1, wt2)
    b2v_ = (b2p0, b2p1, b2p2)
    wov_ = (wo0, wo1, wo2)
    egs = [lax.iota(jnp.int32, 16) + 16 * g for g in range(_CR // 16)]
    d_pk = ekv_[0].at[pl.ds(0, 3 * _CR)]
    d_row = pcv_[0].at[pl.ds(0, 2 * _CR)]
    d_w = wov_[0].at[pl.ds(0, _CR)]

    def _wait(dummy, dst, sem):
        pltpu.make_async_copy(dummy, dst, sem).wait()

    for v in range(_V):
        pltpu.sync_copy(wtv_[v], w2b)
        pltpu.sync_copy(b2v_[v], b2b)
        b2l = b2b[...]
        w2v = [w2b[pl.ds(t * 16, 16)] for t in range(_HP // 16)]

        def _issue_pack(c, j, v=v):
            pltpu.async_copy(
                ekv_[v].at[pl.ds((chunk0 + c) * 3 * _CR, 3 * _CR)],
                pbuf[j], pk[j])

        def _issue_gather(j, v=v):
            pltpu.async_copy(pcv_[v].at[pbuf[j].at[pl.ds(0, 2 * _CR)]],
                             cbuf[j], gp[j])

        for j in range(_SR):
            _issue_pack(j, j)
        _wait(d_pk, pbuf[0], pk[0])
        _issue_gather(0)

        def _group(g, _, v=v):
            for j in range(_SR):
                c = g * _SR + j
                j2 = 1 - j
                _wait(d_row, cbuf[j], gp[j])
                @pl.when(c + 1 < _NCHR)
                def _():
                    _wait(d_pk, pbuf[j2], pk[j2])
                    _issue_gather(j2)

                def _edge(e, _):
                    acc = jnp.zeros((16,), jnp.float32)
                    for t in range(_HP // 16):
                        sgv = cbuf[j][e, pl.ds(t * 16, 16)]
                        dgv = cbuf[j][e + _CR, pl.ds(t * 16, 16)]
                        acc = acc + (jnp.maximum(sgv + dgv, 0.0) * w2v[t])
                    tbuf[e, pl.ds(0, 16)] = acc
                    return 0
                lax.fori_loop(0, _CR, _edge, 0)
                @pl.when(c >= _SR)
                def _():
                    _wait(d_w, wbuf[j], ws[j])
                for gg in range(_CR // 16):
                    tot = jnp.zeros((16,), jnp.float32)
                    for col in range(16):
                        tot = tot + plsc.load_gather(
                            tbuf, [egs[gg], _COLC[col]])
                    dyn = 1.0 / (1.0 + jnp.exp(-(tot + b2l)))
                    scg = plsc.bitcast(
                        pbuf[j][pl.ds(2 * _CR + gg * 16, 16)], jnp.float32)
                    wbuf[j][pl.ds(gg * 16, 16)] = scg * dyn
                base = wid * _EPW + c * _CR
                pltpu.async_copy(wbuf[j], wov_[v].at[pl.ds(base, _CR)], ws[j])
                @pl.when(c + _SR < _NCHR)
                def _():
                    _issue_pack(c + _SR, j)
            return 0
        lax.fori_loop(0, _NCHR // _SR, _group, 0)
        for j in range(_SR):
            _wait(d_w, wbuf[j], ws[j])


def _scat_body(*refs):
    (ya, yb, yc, ek0, ek1, ek2, w0, w1, w2, p0, p1, p2) = refs[:12]
    r = list(refs[12:])
    ybuf = r[0:_S]; pbuf = r[_S:2 * _S]; wbuf = r[2 * _S:3 * _S]
    accsh = r[3 * _S]
    sems = r[3 * _S + 1:]
    gy = sems[0:_S]; pk = sems[_S:2 * _S]; iww = sems[2 * _S:3 * _S]
    scs = sems[3 * _S:4 * _S]; zs = sems[4 * _S]

    cid = lax.axis_index("c")
    sid = lax.axis_index("s")
    wid = sid * _NC + cid
    base0 = wid * _EPW
    chunk0 = wid * _NCHS
    yv_ = (ya, yb, yc)
    ekv_ = (ek0, ek1, ek2)
    wv_ = (w0, w1, w2)
    pv_ = (p0, p1, p2)
    zv = jnp.zeros((16,), jnp.float32)
    egs = [lax.iota(jnp.int32, 16) + 16 * g for g in range(_CS // 16)]
    d_pk = ekv_[0].at[0]
    d_w = wv_[0].at[pl.ds(0, _CS)]
    d_row = yv_[0].at[pl.ds(0, _CS)]
    d_z = yv_[0].at[pl.ds(0, _ZR)]

    def _wait(dummy, dst, sem):
        pltpu.make_async_copy(dummy, dst, sem).wait()

    for v in range(_V):
        def _zrow(rr, _):
            for k in range(_D // 16):
                ybuf[0][rr, pl.ds(k * 16, 16)] = zv
            return 0
        lax.fori_loop(0, _ZR, _zrow, 0)
        for tt in range(_NZT):
            ch = sid + _NS * tt
            @pl.when(ch < _NZ)
            def _():
                pltpu.async_copy(ybuf[0].at[pl.ds(0, _ZR)],
                                 accsh.at[pl.ds(ch * _ZR, _ZR)], zs)
        for tt in range(_NZT):
            ch = sid + _NS * tt
            @pl.when(ch < _NZ)
            def _():
                _wait(d_z, ybuf[0].at[pl.ds(0, _ZR)], zs)
        plsc.subcore_barrier()

        def _issue_idx(c, j, v=v):
            pltpu.async_copy(ekv_[v].at[chunk0 + c], pbuf[j], pk[j])
            pltpu.async_copy(wv_[v].at[pl.ds(base0 + c * _CS, _CS)],
                             wbuf[j], iww[j])

        def _issue_gather(j, v=v):
            pltpu.async_copy(yv_[v].at[pbuf[j].at[0]], ybuf[j], gy[j])

        for j in range(_S):
            _issue_idx(j, j)
        for j in range(_GA_S):
            _wait(d_pk, pbuf[j], pk[j])
            _issue_gather(j)

        def _group(g, _, v=v):
            for j in range(_S):
                c = g * _S + j
                _wait(d_row, ybuf[j], gy[j])
                _wait(d_w, wbuf[j], iww[j])

                def _srow(r16, _):
                    wgv = wbuf[j][pl.ds(r16 * 16, 16)]
                    for k in range(16):
                        rr = r16 * 16 + k
                        wvv = jnp.full((16,), wgv[k])
                        for kk in range(_D // 16):
                            ybuf[j][rr, pl.ds(kk * 16, 16)] = (
                                ybuf[j][rr, pl.ds(kk * 16, 16)] * wvv)
                    return 0
                lax.fori_loop(0, _CS // 16, _srow, 0)
                pltpu.async_copy(ybuf[j], accsh.at[pbuf[j].at[1]], scs[j],
                                 add=True)
                jg = (j + _GA_S) % _S
                @pl.when(c >= _S - _GA_S)
                def _():
                    _wait(d_row, ybuf[jg], scs[jg])
                @pl.when(c + _GA_S < _NCHS)
                def _():
                    _wait(d_pk, pbuf[jg], pk[jg])
                    _issue_gather(jg)
                @pl.when(c + _S < _NCHS)
                def _():
                    _issue_idx(c + _S, j)
            return 0
        lax.fori_loop(0, _NCHS // _S, _group, 0)
        for dd in range(_S - _GA_S):
            j = (_NCHS - 1 - dd) % _S
            _wait(d_row, ybuf[j], scs[j])
        plsc.subcore_barrier()
        for tt in range(_NZT):
            ch = sid + _NS * tt
            @pl.when(ch < _NZ)
            def _():
                pltpu.sync_copy(accsh.at[pl.ds(ch * _ZR, _ZR)],
                                pv_[v].at[pl.ds(cid * _N + ch * _ZR, _ZR)])
        plsc.subcore_barrier()


def _full(i):
    return (0,) * i


def kernel(x_proj, edge_indices, edge_scores, cancer_type_id, causal_scores,
           cancer_table, rw_W1_0, rw_b1_0, rw_W2_0, rw_b2_0, lin1_0, lin2_0,
           rw_W1_1, rw_b1_1, rw_W2_1, rw_b2_1, lin1_1, lin2_1,
           rw_W1_2, rw_b1_2, rw_W2_2, rw_b2_2, lin1_2, lin2_2):
    f32 = jnp.float32
    W1 = jnp.stack([rw_W1_0, rw_W1_1, rw_W1_2])
    W1 = jnp.pad(W1, ((0, 0), (0, 0), (0, _HP - _HID)))
    b1 = jnp.pad(jnp.stack([rw_b1_0, rw_b1_1, rw_b1_2]),
                 ((0, 0), (0, _HP - _HID)))[:, None, :]
    W2 = jnp.pad(jnp.stack([rw_W2_0, rw_W2_1, rw_W2_2])[..., 0],
                 ((0, 0), (0, _HP - _HID)))
    b2 = jnp.broadcast_to(jnp.stack([rw_b2_0, rw_b2_1, rw_b2_2]), (_V, 16))
    w1a = W1[:, :_D]
    w1b = W1[:, _D:2 * _D]
    w1c = W1[:, 2 * _D:2 * _D + 32]
    uvec = (W1[:, 2 * _D + 32] + W1[:, 2 * _D + 34])[:, None, :]
    tvec = (W1[:, 2 * _D + 33] - W1[:, 2 * _D + 34])[:, None, :]
    lin1 = jnp.stack([lin1_0, lin1_1, lin1_2])
    lin2 = jnp.stack([lin2_0, lin2_1, lin2_2])
    cid = cancer_type_id.astype(jnp.int32)

    pad = _EP - _E
    se = jnp.pad(edge_indices[:, 0, :], ((0, 0), (0, pad)))
    de = jnp.pad(edge_indices[:, 1, :], ((0, 0), (0, pad)))
    sc = jnp.pad(edge_scores, ((0, 0), (0, pad)))

    full = lambda shape: pl.BlockSpec(shape, lambda i: _full(len(shape)))
    tc1 = pl.pallas_call(
        _tc1_body,
        grid=(_GB,),
        in_specs=[
            pl.BlockSpec((_BR, _D), lambda i: (i, 0)),
            pl.BlockSpec((_BR, 1), lambda i: (i, 0)),
            full((16, 32)),
            pl.BlockSpec(memory_space=pltpu.SMEM),
            full((_V, _D, _HP)),
            full((_V, _D, _HP)),
            full((_V, 32, _HP)),
            full((_V, 1, _HP)),
            full((_V, 1, _HP)),
            full((_V, 1, _HP)),
            full((_V, _D, _D)),
        ],
        out_specs=[
            pl.BlockSpec((_V, _BR, _HP), lambda i: (0, i, 0)),
            pl.BlockSpec((_V, _BR, _HP), lambda i: (0, i, 0)),
            pl.BlockSpec((_V, _BR, _D), lambda i: (0, i, 0)),
            pl.BlockSpec((_BR, _D), lambda i: (i, 0)),
        ],
        out_shape=[
            jax.ShapeDtypeStruct((_V, _N, _HP), f32),
            jax.ShapeDtypeStruct((_V, _N, _HP), f32),
            jax.ShapeDtypeStruct((_V, _N, _D), f32),
            jax.ShapeDtypeStruct((_N, _D), f32),
        ],
    )
    ps, pd, y1, xh = tc1(x_proj, causal_scores, cancer_table, cid,
                         w1a, w1b, w1c, uvec, tvec, b1, lin1)

    pre_cat = jnp.concatenate([ps, pd], axis=1)
    i32 = jnp.int32
    se_r = se.reshape(_V, _NW * _NCHR, _CR)
    de_r = de.reshape(_V, _NW * _NCHR, _CR)
    sc_r = lax.bitcast_convert_type(sc, i32).reshape(_V, _NW * _NCHR, _CR)
    ekR = jnp.concatenate([se_r, de_r + _N, sc_r], axis=2).reshape(_V, -1)
    ekS = jnp.stack([se.reshape(_V, _NW * _NCHS, _CS),
                     de.reshape(_V, _NW * _NCHS, _CS)], axis=2)

    mesh = plsc.VectorSubcoreMesh(core_axis_name="c", subcore_axis_name="s")
    sc_params = pltpu.CompilerParams(needs_layout_passes=False)
    rw = pl.kernel(
        _rw_body,
        compiler_params=sc_params,
        out_type=[jax.ShapeDtypeStruct((_EP,), f32)] * 3,
        mesh=mesh,
        scratch_types=(
            [pltpu.VMEM((2 * _CR, _HP), f32)] * _SR
            + [pltpu.VMEM((3 * _CR,), i32)] * _SR
            + [pltpu.VMEM((_CR,), f32)] * _SR
            + [pltpu.VMEM((_HP,), f32),
               pltpu.VMEM((16,), f32),
               pltpu.VMEM((_CR, 17), f32)]
            + [pltpu.SemaphoreType.DMA] * (3 * _SR)
        ),
    )
    w0, w1_, w2_ = rw(pre_cat[0], pre_cat[1], pre_cat[2],
                      ekR[0], ekR[1], ekR[2],
                      W2[0], W2[1], W2[2], b2[0], b2[1], b2[2])

    scat_scratch = (
        [pltpu.VMEM((_CS, _D), f32)] * _S
        + [pltpu.VMEM((2, _CS), i32)] * _S
        + [pltpu.VMEM((_CS,), f32)] * _S
        + [pltpu.VMEM_SHARED((_N, _D), f32)]
        + [pltpu.SemaphoreType.DMA] * (4 * _S + 1)
    )
    scat1 = pl.kernel(
        _scat_body,
        compiler_params=sc_params,
        out_type=[jax.ShapeDtypeStruct((2 * _N, _D), f32)] * 3,
        mesh=mesh,
        scratch_types=scat_scratch,
    )
    q0, q1, q2 = scat1(y1[0], y1[1], y1[2],
                       ekS[0], ekS[1], ekS[2],
                       w0, w1_, w2_)

    half_a = pl.BlockSpec((_BR, _D), lambda i: (i, 0))
    half_b = pl.BlockSpec((_BR, _D), lambda i: (i + _GB, 0))
    tc2 = pl.pallas_call(
        _tc2_body,
        grid=(_GB,),
        in_specs=[half_a, half_b, half_a, half_b, half_a, half_b,
                  pl.BlockSpec((_V, _BR, _D), lambda i: (0, i, 0)),
                  pl.BlockSpec((_BR, _D), lambda i: (i, 0)),
                  full((_V, _D, _D))],
        out_specs=[pl.BlockSpec((_V, _BR, _D), lambda i: (0, i, 0)),
                   pl.BlockSpec((_V, _BR, _D), lambda i: (0, i, 0))],
        out_shape=[jax.ShapeDtypeStruct((_V, _N, _D), f32),
                   jax.ShapeDtypeStruct((_V, _N, _D), f32)],
    )
    h1, y2 = tc2(q0, q0, q1, q1, q2, q2, y1, xh, lin2)

    scat2 = pl.kernel(
        _scat_body,
        compiler_params=sc_params,
        out_type=[jax.ShapeDtypeStruct((2 * _N, _D), f32)] * 3,
        mesh=mesh,
        scratch_types=scat_scratch,
    )
    r0, r1, r2 = scat2(y2[0], y2[1], y2[2],
                       ekS[0], ekS[1], ekS[2],
                       w0, w1_, w2_)

    tc3 = pl.pallas_call(
        _tc3_body,
        grid=(_GB,),
        in_specs=[half_a, half_b, half_a, half_b, half_a, half_b,
                  pl.BlockSpec((_V, _BR, _D), lambda i: (0, i, 0)),
                  pl.BlockSpec((_V, _BR, _D), lambda i: (0, i, 0))],
        out_specs=[pl.BlockSpec((_BR, _V * _D), lambda i: (i, 0)),
                   pl.BlockSpec((_V, _BR, _D), lambda i: (0, i, 0))],
        out_shape=[jax.ShapeDtypeStruct((_N, _V * _D), f32),
                   jax.ShapeDtypeStruct((_V, _N, _D), f32)],
    )
    out, h2 = tc3(r0, r0, r1, r1, r2, r2, y2, h1)
    return (out, h2[0], h2[1], h2[2])
